# Initial kernel scaffold; baseline (speedup 1.0000x reference)
import jax
import jax.numpy as jnp
from jax.experimental import pallas as pl


def kernel(x, edge_index, edge_weight, W1, b1, W2, b2, W3, b3):
    raise NotImplementedError("write your pallas kernel here")


# speedup vs baseline: 1.2541x; 1.2541x over previous
import functools

import jax
import jax.numpy as jnp
from jax.experimental import pallas as pl

_MB = 512


def _mm_body(x_ref, w_ref, b_ref, o_ref, *, act):
    acc = jnp.dot(x_ref[...], w_ref[...], preferred_element_type=jnp.float32)
    acc = acc + b_ref[...]
    if act:
        acc = jnp.where(acc > 0, acc, jnp.exp(jnp.minimum(acc, 0.0)) - 1.0)
    o_ref[...] = acc


def _mm(x, w, b, act):
    M, K = x.shape
    Nout = w.shape[1]
    return pl.pallas_call(
        functools.partial(_mm_body, act=act),
        grid=(M // _MB,),
        in_specs=[
            pl.BlockSpec((_MB, K), lambda i: (i, 0)),
            pl.BlockSpec((K, Nout), lambda i: (0, 0)),
            pl.BlockSpec((1, Nout), lambda i: (0, 0)),
        ],
        out_specs=pl.BlockSpec((_MB, Nout), lambda i: (i, 0)),
        out_shape=jax.ShapeDtypeStruct((M, Nout), jnp.float32),
    )(x, w, b.reshape(1, Nout))


def kernel(x, edge_index, edge_weight, W1, b1, W2, b2, W3, b3):
    N = x.shape[0]
    Np = 10240
    K1 = W1.shape[0]
    H = W1.shape[2]
    F = x.shape[1]

    row, col = edge_index[0], edge_index[1]
    deg = jnp.zeros((N,), edge_weight.dtype).at[col].add(edge_weight)
    dis = jnp.where(deg > 0, jax.lax.rsqrt(jnp.maximum(deg, 1e-30)), 0.0)
    norm = dis[row] * edge_weight * dis[col]

    def hop(v):
        msg = norm[:, None] * jnp.take(v, row, axis=0)
        return jnp.zeros((N, v.shape[1]), v.dtype).at[col].add(msg)

    hops1 = [x]
    for _ in range(K1 - 1):
        hops1.append(hop(hops1[-1]))
    X1 = jnp.concatenate(hops1, axis=1)
    X1 = jnp.pad(X1, ((0, Np - N), (0, 128 - K1 * F)))
    W1s = jnp.pad(W1.reshape(K1 * F, H), ((0, 128 - K1 * F), (0, 0)))
    h1 = _mm(X1, W1s, b1, act=True)

    h1n = h1[:N]
    hops2 = [h1n]
    for _ in range(K1 - 1):
        hops2.append(hop(hops2[-1]))
    X2 = jnp.pad(jnp.concatenate(hops2, axis=1), ((0, Np - N), (0, 0)))
    W2s = W2.reshape(K1 * H, H)
    h2 = _mm(X2, W2s, b2, act=True)

    W3s = jnp.pad(W3.transpose(1, 0, 2).reshape(H, K1 * F), ((0, 0), (0, 128 - K1 * F)))
    b3p = jnp.pad(b3, (0, 128 - F))
    y = _mm(h2, W3s, b3p, act=False)[:N]
    ys = [y[:, k * F:(k + 1) * F] for k in range(K1)]
    t = ys[K1 - 1]
    for k in range(K1 - 2, -1, -1):
        t = ys[k] + hop(t)
    return t


# speedup vs baseline: 1.3842x; 1.1037x over previous
import functools

import jax
import jax.numpy as jnp
from jax import lax
from jax.experimental import pallas as pl
from jax.experimental.pallas import tpu as pltpu
from jax.experimental.pallas import tpu_sc as plsc

_MB = 512
_N = 10000
_E = 160000
_NT = 16
_G = 128
_EWP = 10240
_NCH = _EWP // _G


def _mm_body(x_ref, w_ref, b_ref, o_ref, *, act):
    acc = jnp.dot(x_ref[...], w_ref[...], preferred_element_type=jnp.float32)
    acc = acc + b_ref[...]
    if act:
        acc = jnp.where(acc > 0, acc, jnp.exp(jnp.minimum(acc, 0.0)) - 1.0)
    o_ref[...] = acc


def _mm(x, w, b, act):
    M, K = x.shape
    Nout = w.shape[1]
    return pl.pallas_call(
        functools.partial(_mm_body, act=act),
        grid=(M // _MB,),
        in_specs=[
            pl.BlockSpec((_MB, K), lambda i: (i, 0)),
            pl.BlockSpec((K, Nout), lambda i: (0, 0)),
            pl.BlockSpec((1, Nout), lambda i: (0, 0)),
        ],
        out_specs=pl.BlockSpec((_MB, Nout), lambda i: (i, 0)),
        out_shape=jax.ShapeDtypeStruct((M, Nout), jnp.float32),
    )(x, w, b.reshape(1, Nout))


_CH = 64
_NQ = 512 // _CH
_KV = _CH // 16


def _hop512_body(src, zsrc, row3, col3, norm3, out,
                 rowv, colv, normv, gbuf, acc, sem):
    c = lax.axis_index("c")
    s = lax.axis_index("s")
    last = s == _NT - 1

    pltpu.sync_copy(row3.at[s], rowv)
    pltpu.sync_copy(col3.at[s], colv)
    pltpu.sync_copy(norm3.at[s], normv)

    def chunk_loop(q):
        def chunk(i, carry):
            pltpu.async_copy(src.at[q].at[rowv.at[i]], gbuf, sem).wait()

            def edge16(eb, c2_):
                nvec = normv[i, pl.ds(eb * 16, 16)]
                for j in range(16):
                    nb = lax.broadcast(nvec[j], (16,))
                    e = eb * 16 + j
                    for k in range(_KV):
                        sl = pl.ds(k * 16, 16)
                        gbuf[e, sl] = gbuf[e, sl] * nb
                return c2_
            lax.fori_loop(0, _G // 16, edge16, 0)
            pltpu.sync_copy(gbuf, acc.at[colv.at[i]], add=True)
            return carry
        lax.fori_loop(0, _NCH, chunk, 0)

    for p in range(_NQ // 2):
        pl.when(jnp.logical_not(last))(
            lambda: pltpu.sync_copy(zsrc.at[pl.ds(s * 624, 624)],
                                    acc.at[pl.ds(s * 624, 624)]))
        pl.when(last)(
            lambda: pltpu.sync_copy(zsrc.at[pl.ds(9360, 640)],
                                    acc.at[pl.ds(9360, 640)]))
        plsc.subcore_barrier()
        for cv in range(2):
            pl.when(c == cv)(functools.partial(chunk_loop, (_NQ // 2) * cv + p))
        plsc.subcore_barrier()
        for cv in range(2):
            q = (_NQ // 2) * cv + p
            pl.when(jnp.logical_and(c == cv, jnp.logical_not(last)))(
                functools.partial(
                    lambda qq: pltpu.sync_copy(
                        acc.at[pl.ds(s * 624, 624)],
                        out.at[qq].at[pl.ds(s * 624, 624)]), q))
            pl.when(jnp.logical_and(c == cv, last))(
                functools.partial(
                    lambda qq: pltpu.sync_copy(
                        acc.at[pl.ds(9360, 640)],
                        out.at[qq].at[pl.ds(9360, 640)]), q))
        plsc.subcore_barrier()


_hop512_call = pl.kernel(
    _hop512_body,
    out_type=jax.ShapeDtypeStruct((_NQ, _N, _CH), jnp.float32),
    mesh=plsc.VectorSubcoreMesh(core_axis_name="c", subcore_axis_name="s"),
    compiler_params=pltpu.CompilerParams(use_tc_tiling_on_sc=False),
    scratch_types=[
        pltpu.VMEM((_NCH, _G), jnp.int32),
        pltpu.VMEM((_NCH, _G), jnp.int32),
        pltpu.VMEM((_NCH, _G), jnp.float32),
        pltpu.VMEM((_G, _CH), jnp.float32),
        pltpu.VMEM_SHARED((_N, _CH), jnp.float32),
        pltpu.SemaphoreType.DMA,
    ],
)


def kernel(x, edge_index, edge_weight, W1, b1, W2, b2, W3, b3):
    N = x.shape[0]
    Np = 10240
    K1 = W1.shape[0]
    H = W1.shape[2]
    F = x.shape[1]

    row, col = edge_index[0], edge_index[1]
    deg = jnp.zeros((N,), edge_weight.dtype).at[col].add(edge_weight)
    dis = jnp.where(deg > 0, jax.lax.rsqrt(jnp.maximum(deg, 1e-30)), 0.0)
    norm = dis[row] * edge_weight * dis[col]

    def _meta(v):
        return jnp.pad(v.reshape(_NT, _E // _NT), ((0, 0), (0, _EWP - _E // _NT))
                       ).reshape(_NT, _NCH, _G)
    row3 = _meta(row)
    col3 = _meta(col)
    norm3 = _meta(norm)

    def hop(v):
        msg = norm[:, None] * jnp.take(v, row, axis=0)
        return jnp.zeros((N, v.shape[1]), v.dtype).at[col].add(msg)

    def hop512_pack(v):
        return jnp.transpose(v.reshape(_N, _NQ, _CH), (1, 0, 2))

    zsrc = jnp.zeros((_N, _CH), jnp.float32)

    def hop512(src):
        o = _hop512_call(src, zsrc, row3, col3, norm3)
        flat = jnp.transpose(o, (1, 0, 2)).reshape(_N, 512)
        return o, flat

    hops1 = [x]
    for _ in range(K1 - 1):
        hops1.append(hop(hops1[-1]))
    X1 = jnp.concatenate(hops1, axis=1)
    X1 = jnp.pad(X1, ((0, Np - N), (0, 128 - K1 * F)))
    W1s = jnp.pad(W1.reshape(K1 * F, H), ((0, 128 - K1 * F), (0, 0)))
    h1 = _mm(X1, W1s, b1, act=True)

    h1n = h1[:N]
    hops2 = [h1n]
    src = hop512_pack(h1n)
    for _ in range(K1 - 1):
        src, flat = hop512(src)
        hops2.append(flat)
    X2 = jnp.pad(jnp.concatenate(hops2, axis=1), ((0, Np - N), (0, 0)))
    W2s = W2.reshape(K1 * H, H)
    h2 = _mm(X2, W2s, b2, act=True)

    W3s = jnp.pad(W3.transpose(1, 0, 2).reshape(H, K1 * F), ((0, 0), (0, 128 - K1 * F)))
    b3p = jnp.pad(b3, (0, 128 - F))
    y = _mm(h2, W3s, b3p, act=False)[:N]
    ys = [y[:, k * F:(k + 1) * F] for k in range(K1)]
    t = ys[K1 - 1]
    for k in range(K1 - 2, -1, -1):
        t = ys[k] + hop(t)
    return t


# speedup vs baseline: 1.4450x; 1.0439x over previous
import functools

import jax
import jax.numpy as jnp
from jax import lax
from jax.experimental import pallas as pl
from jax.experimental.pallas import tpu as pltpu
from jax.experimental.pallas import tpu_sc as plsc

_MB = 512
_N = 10000
_E = 160000
_NT = 16
_G = 128
_EWP = 10240
_NCH = _EWP // _G


def _mm_body(x_ref, w_ref, b_ref, o_ref, *, act):
    acc = jnp.dot(x_ref[...], w_ref[...], preferred_element_type=jnp.float32)
    acc = acc + b_ref[...]
    if act:
        acc = jnp.where(acc > 0, acc, jnp.exp(jnp.minimum(acc, 0.0)) - 1.0)
    o_ref[...] = acc


def _mm(x, w, b, act):
    M, K = x.shape
    Nout = w.shape[1]
    return pl.pallas_call(
        functools.partial(_mm_body, act=act),
        grid=(M // _MB,),
        in_specs=[
            pl.BlockSpec((_MB, K), lambda i: (i, 0)),
            pl.BlockSpec((K, Nout), lambda i: (0, 0)),
            pl.BlockSpec((1, Nout), lambda i: (0, 0)),
        ],
        out_specs=pl.BlockSpec((_MB, Nout), lambda i: (i, 0)),
        out_shape=jax.ShapeDtypeStruct((M, Nout), jnp.float32),
    )(x, w, b.reshape(1, Nout))


_CH = 64
_NQ = 512 // _CH
_KV = _CH // 16


def _hop512_body(src, zsrc, row3, col3, norm3, out,
                 rowv, colv, normv, gbuf0, gbuf1, acc,
                 gsem0, gsem1, ssem0, ssem1):
    c = lax.axis_index("c")
    s = lax.axis_index("s")
    last = s == _NT - 1

    pltpu.sync_copy(row3.at[s], rowv)
    pltpu.sync_copy(col3.at[s], colv)
    pltpu.sync_copy(norm3.at[s], normv)

    gbufs = (gbuf0, gbuf1)
    gsems = (gsem0, gsem1)
    ssems = (ssem0, ssem1)

    def chunk_loop(q):
        def compute(i, buf):
            def edge16(eb, c2_):
                nvec = normv[i, pl.ds(eb * 16, 16)]
                for j in range(16):
                    nb = lax.broadcast(nvec[j], (16,))
                    e = eb * 16 + j
                    for k in range(_KV):
                        sl = pl.ds(k * 16, 16)
                        buf[e, sl] = buf[e, sl] * nb
                return c2_
            lax.fori_loop(0, _G // 16, edge16, 0)

        def body2(i2, carry):
            ia = 2 * i2
            ib = 2 * i2 + 1
            da = pltpu.async_copy(src.at[q].at[rowv.at[ia]], gbuf0, gsem0)
            db = pltpu.async_copy(src.at[q].at[rowv.at[ib]], gbuf1, gsem1)
            da.wait()
            compute(ia, gbuf0)
            sa = pltpu.async_copy(gbuf0, acc.at[colv.at[ia]], ssem0, add=True)
            db.wait()
            compute(ib, gbuf1)
            sb = pltpu.async_copy(gbuf1, acc.at[colv.at[ib]], ssem1, add=True)
            sa.wait()
            sb.wait()
            return carry
        lax.fori_loop(0, _NCH // 2, body2, 0)

    for p in range(_NQ // 2):
        pl.when(jnp.logical_not(last))(
            lambda: pltpu.sync_copy(zsrc.at[pl.ds(s * 624, 624)],
                                    acc.at[pl.ds(s * 624, 624)]))
        pl.when(last)(
            lambda: pltpu.sync_copy(zsrc.at[pl.ds(9360, 640)],
                                    acc.at[pl.ds(9360, 640)]))
        plsc.subcore_barrier()
        for cv in range(2):
            pl.when(c == cv)(functools.partial(chunk_loop, (_NQ // 2) * cv + p))
        plsc.subcore_barrier()
        for cv in range(2):
            q = (_NQ // 2) * cv + p
            pl.when(jnp.logical_and(c == cv, jnp.logical_not(last)))(
                functools.partial(
                    lambda qq: pltpu.sync_copy(
                        acc.at[pl.ds(s * 624, 624)],
                        out.at[qq].at[pl.ds(s * 624, 624)]), q))
            pl.when(jnp.logical_and(c == cv, last))(
                functools.partial(
                    lambda qq: pltpu.sync_copy(
                        acc.at[pl.ds(9360, 640)],
                        out.at[qq].at[pl.ds(9360, 640)]), q))
        plsc.subcore_barrier()


_hop512_call = pl.kernel(
    _hop512_body,
    out_type=jax.ShapeDtypeStruct((_NQ, _N, _CH), jnp.float32),
    mesh=plsc.VectorSubcoreMesh(core_axis_name="c", subcore_axis_name="s"),
    compiler_params=pltpu.CompilerParams(use_tc_tiling_on_sc=False),
    scratch_types=[
        pltpu.VMEM((_NCH, _G), jnp.int32),
        pltpu.VMEM((_NCH, _G), jnp.int32),
        pltpu.VMEM((_NCH, _G), jnp.float32),
        pltpu.VMEM((_G, _CH), jnp.float32),
        pltpu.VMEM((_G, _CH), jnp.float32),
        pltpu.VMEM_SHARED((_N, _CH), jnp.float32),
        pltpu.SemaphoreType.DMA,
        pltpu.SemaphoreType.DMA,
        pltpu.SemaphoreType.DMA,
        pltpu.SemaphoreType.DMA,
    ],
)


def kernel(x, edge_index, edge_weight, W1, b1, W2, b2, W3, b3):
    N = x.shape[0]
    Np = 10240
    K1 = W1.shape[0]
    H = W1.shape[2]
    F = x.shape[1]

    row, col = edge_index[0], edge_index[1]
    deg = jnp.zeros((N,), edge_weight.dtype).at[col].add(edge_weight)
    dis = jnp.where(deg > 0, jax.lax.rsqrt(jnp.maximum(deg, 1e-30)), 0.0)
    norm = dis[row] * edge_weight * dis[col]

    def _meta(v):
        return jnp.pad(v.reshape(_NT, _E // _NT), ((0, 0), (0, _EWP - _E // _NT))
                       ).reshape(_NT, _NCH, _G)
    row3 = _meta(row)
    col3 = _meta(col)
    norm3 = _meta(norm)

    def hop(v):
        msg = norm[:, None] * jnp.take(v, row, axis=0)
        return jnp.zeros((N, v.shape[1]), v.dtype).at[col].add(msg)

    def hop512_pack(v):
        return jnp.transpose(v.reshape(_N, _NQ, _CH), (1, 0, 2))

    zsrc = jnp.zeros((_N, _CH), jnp.float32)

    def hop512(src):
        o = _hop512_call(src, zsrc, row3, col3, norm3)
        flat = jnp.transpose(o, (1, 0, 2)).reshape(_N, 512)
        return o, flat

    hops1 = [x]
    for _ in range(K1 - 1):
        hops1.append(hop(hops1[-1]))
    X1 = jnp.concatenate(hops1, axis=1)
    X1 = jnp.pad(X1, ((0, Np - N), (0, 128 - K1 * F)))
    W1s = jnp.pad(W1.reshape(K1 * F, H), ((0, 128 - K1 * F), (0, 0)))
    h1 = _mm(X1, W1s, b1, act=True)

    h1n = h1[:N]
    hops2 = [h1n]
    src = hop512_pack(h1n)
    for _ in range(K1 - 1):
        src, flat = hop512(src)
        hops2.append(flat)
    X2 = jnp.pad(jnp.concatenate(hops2, axis=1), ((0, Np - N), (0, 0)))
    W2s = W2.reshape(K1 * H, H)
    h2 = _mm(X2, W2s, b2, act=True)

    W3s = jnp.pad(W3.transpose(1, 0, 2).reshape(H, K1 * F), ((0, 0), (0, 128 - K1 * F)))
    b3p = jnp.pad(b3, (0, 128 - F))
    y = _mm(h2, W3s, b3p, act=False)[:N]
    ys = [y[:, k * F:(k + 1) * F] for k in range(K1)]
    t = ys[K1 - 1]
    for k in range(K1 - 2, -1, -1):
        t = ys[k] + hop(t)
    return t


# speedup vs baseline: 3.4926x; 2.4170x over previous
import functools

import jax
import jax.numpy as jnp
from jax import lax
from jax.experimental import pallas as pl
from jax.experimental.pallas import tpu as pltpu
from jax.experimental.pallas import tpu_sc as plsc

_MB = 512
_N = 10000
_E = 160000
_NT = 16
_G = 128
_EWP = 10240
_NCH = _EWP // _G


def _mm_body(x_ref, w_ref, b_ref, o_ref, *, act):
    acc = jnp.dot(x_ref[...], w_ref[...], preferred_element_type=jnp.float32)
    acc = acc + b_ref[...]
    if act:
        acc = jnp.where(acc > 0, acc, jnp.exp(jnp.minimum(acc, 0.0)) - 1.0)
    o_ref[...] = acc


def _mm(x, w, b, act):
    M, K = x.shape
    Nout = w.shape[1]
    return pl.pallas_call(
        functools.partial(_mm_body, act=act),
        grid=(M // _MB,),
        in_specs=[
            pl.BlockSpec((_MB, K), lambda i: (i, 0)),
            pl.BlockSpec((K, Nout), lambda i: (0, 0)),
            pl.BlockSpec((1, Nout), lambda i: (0, 0)),
        ],
        out_specs=pl.BlockSpec((_MB, Nout), lambda i: (i, 0)),
        out_shape=jax.ShapeDtypeStruct((M, Nout), jnp.float32),
    )(x, w, b.reshape(1, Nout))


_CH = 64
_NQ = 512 // _CH
_KV = _CH // 16


def _hop512_body(src, zsrc, row3, col3, norm3, out,
                 rowv, colv, normv, gbuf0, gbuf1, acc,
                 gsem0, gsem1, ssem0, ssem1):
    c = lax.axis_index("c")
    s = lax.axis_index("s")
    last = s == _NT - 1

    pltpu.sync_copy(row3.at[s], rowv)
    pltpu.sync_copy(col3.at[s], colv)
    pltpu.sync_copy(norm3.at[s], normv)

    gbufs = (gbuf0, gbuf1)
    gsems = (gsem0, gsem1)
    ssems = (ssem0, ssem1)

    def chunk_loop(q):
        def compute(i, buf):
            def edge16(eb, c2_):
                nvec = normv[i, pl.ds(eb * 16, 16)]
                for j in range(16):
                    nb = lax.broadcast(nvec[j], (16,))
                    e = eb * 16 + j
                    for k in range(_KV):
                        sl = pl.ds(k * 16, 16)
                        buf[e, sl] = buf[e, sl] * nb
                return c2_
            lax.fori_loop(0, _G // 16, edge16, 0)

        def body2(i2, carry):
            ia = 2 * i2
            ib = 2 * i2 + 1
            da = pltpu.async_copy(src.at[q].at[rowv.at[ia]], gbuf0, gsem0)
            db = pltpu.async_copy(src.at[q].at[rowv.at[ib]], gbuf1, gsem1)
            da.wait()
            compute(ia, gbuf0)
            sa = pltpu.async_copy(gbuf0, acc.at[colv.at[ia]], ssem0, add=True)
            db.wait()
            compute(ib, gbuf1)
            sb = pltpu.async_copy(gbuf1, acc.at[colv.at[ib]], ssem1, add=True)
            sa.wait()
            sb.wait()
            return carry
        lax.fori_loop(0, _NCH // 2, body2, 0)

    for p in range(_NQ // 2):
        pl.when(jnp.logical_not(last))(
            lambda: pltpu.sync_copy(zsrc.at[pl.ds(s * 624, 624)],
                                    acc.at[pl.ds(s * 624, 624)]))
        pl.when(last)(
            lambda: pltpu.sync_copy(zsrc.at[pl.ds(9360, 640)],
                                    acc.at[pl.ds(9360, 640)]))
        plsc.subcore_barrier()
        for cv in range(2):
            pl.when(c == cv)(functools.partial(chunk_loop, (_NQ // 2) * cv + p))
        plsc.subcore_barrier()
        for cv in range(2):
            q = (_NQ // 2) * cv + p
            pl.when(jnp.logical_and(c == cv, jnp.logical_not(last)))(
                functools.partial(
                    lambda qq: pltpu.sync_copy(
                        acc.at[pl.ds(s * 624, 624)],
                        out.at[qq].at[pl.ds(s * 624, 624)]), q))
            pl.when(jnp.logical_and(c == cv, last))(
                functools.partial(
                    lambda qq: pltpu.sync_copy(
                        acc.at[pl.ds(9360, 640)],
                        out.at[qq].at[pl.ds(9360, 640)]), q))
        plsc.subcore_barrier()


_hop512_call = pl.kernel(
    _hop512_body,
    out_type=jax.ShapeDtypeStruct((_NQ, _N, _CH), jnp.float32),
    mesh=plsc.VectorSubcoreMesh(core_axis_name="c", subcore_axis_name="s"),
    compiler_params=pltpu.CompilerParams(use_tc_tiling_on_sc=False),
    scratch_types=[
        pltpu.VMEM((_NCH, _G), jnp.int32),
        pltpu.VMEM((_NCH, _G), jnp.int32),
        pltpu.VMEM((_NCH, _G), jnp.float32),
        pltpu.VMEM((_G, _CH), jnp.float32),
        pltpu.VMEM((_G, _CH), jnp.float32),
        pltpu.VMEM_SHARED((_N, _CH), jnp.float32),
        pltpu.SemaphoreType.DMA,
        pltpu.SemaphoreType.DMA,
        pltpu.SemaphoreType.DMA,
        pltpu.SemaphoreType.DMA,
    ],
)


def _small_chunk_loop(cur, tgt, rowv, colv, normv, g0, g1, s0, s1, q0, q1):
    def compute(i, buf):
        def edge16(eb, c2_):
            nvec = normv[i, pl.ds(eb * 16, 16)]
            for j in range(16):
                nb = lax.broadcast(nvec[j], (16,))
                e = eb * 16 + j
                buf[e, :] = buf[e, :] * nb
            return c2_
        lax.fori_loop(0, _G // 16, edge16, 0)

    def body2(i2, carry):
        ia = 2 * i2
        ib = 2 * i2 + 1
        da = pltpu.async_copy(cur.at[rowv.at[ia]], g0, q0)
        db = pltpu.async_copy(cur.at[rowv.at[ib]], g1, q1)
        da.wait()
        compute(ia, g0)
        sa = pltpu.async_copy(g0, tgt.at[colv.at[ia]], s0, add=True)
        db.wait()
        compute(ib, g1)
        sb = pltpu.async_copy(g1, tgt.at[colv.at[ib]], s1, add=True)
        sa.wait()
        sb.wait()
        return carry
    lax.fori_loop(0, _NCH // 2, body2, 0)


def _dis_body(deg_ref, o_ref):
    d = deg_ref[...]
    o_ref[...] = jnp.where(d > 0, lax.rsqrt(jnp.maximum(d, 1e-30)), 0.0)


_dis_call = pl.pallas_call(
    _dis_body,
    out_shape=jax.ShapeDtypeStruct((_EWP // _G, _G), jnp.float32),
)


def _deg_body(zsrc1d, col3, w3, deg_o, colv, wv, acc1, q0, q1):
    c = lax.axis_index("c")
    s = lax.axis_index("s")
    last = s == _NT - 1

    pltpu.sync_copy(col3.at[s], colv)
    pltpu.sync_copy(w3.at[s], wv)

    def both(fn_main, fn_last):
        pl.when(jnp.logical_not(last))(fn_main)
        pl.when(last)(fn_last)

    both(lambda: pltpu.sync_copy(zsrc1d.at[pl.ds(s * 624, 624)],
                                 acc1.at[pl.ds(s * 624, 624)]),
         lambda: pltpu.sync_copy(zsrc1d.at[pl.ds(9360, 640)],
                                 acc1.at[pl.ds(9360, 640)]))
    plsc.subcore_barrier()

    def body2(i2, carry):
        sa = pltpu.async_copy(wv.at[2 * i2], acc1.at[colv.at[2 * i2]],
                              q0, add=True)
        sb = pltpu.async_copy(wv.at[2 * i2 + 1], acc1.at[colv.at[2 * i2 + 1]],
                              q1, add=True)
        sa.wait()
        sb.wait()
        return carry
    lax.fori_loop(0, _NCH // 2, body2, 0)
    plsc.subcore_barrier()
    pl.when(jnp.logical_and(c == 0, jnp.logical_not(last)))(
        lambda: pltpu.sync_copy(acc1.at[pl.ds(s * 624, 624)],
                                deg_o.at[pl.ds(s * 624, 624)]))
    pl.when(jnp.logical_and(c == 0, last))(
        lambda: pltpu.sync_copy(acc1.at[pl.ds(9360, 640)],
                                deg_o.at[pl.ds(9360, 640)]))


_deg_call = pl.kernel(
    _deg_body,
    out_type=jax.ShapeDtypeStruct((_N,), jnp.float32),
    mesh=plsc.VectorSubcoreMesh(core_axis_name="c", subcore_axis_name="s"),
    compiler_params=pltpu.CompilerParams(use_tc_tiling_on_sc=False),
    scratch_types=[
        pltpu.VMEM((_NCH, _G), jnp.int32),
        pltpu.VMEM((_NCH, _G), jnp.float32),
        pltpu.VMEM_SHARED((_N,), jnp.float32),
        pltpu.SemaphoreType.DMA,
        pltpu.SemaphoreType.DMA,
    ],
)


def _l1_body(x16, zsrc16, dis, row3, col3, w3, outs, norm3o,
             rowv, colv, wv, normv, g0, g1, d0, d1,
             curA, accB, q0, q1, s0, s1):
    c = lax.axis_index("c")
    s = lax.axis_index("s")
    last = s == _NT - 1

    pltpu.sync_copy(row3.at[s], rowv)
    pltpu.sync_copy(col3.at[s], colv)
    pltpu.sync_copy(w3.at[s], wv)

    def both(fn_main, fn_last):
        pl.when(jnp.logical_not(last))(fn_main)
        pl.when(last)(fn_last)

    both(lambda: pltpu.sync_copy(x16.at[pl.ds(s * 624, 624)],
                                 curA.at[pl.ds(s * 624, 624)]),
         lambda: pltpu.sync_copy(x16.at[pl.ds(9360, 640)],
                                 curA.at[pl.ds(9360, 640)]))
    both(lambda: pltpu.sync_copy(zsrc16.at[pl.ds(s * 624, 624)],
                                 accB.at[pl.ds(s * 624, 624)]),
         lambda: pltpu.sync_copy(zsrc16.at[pl.ds(9360, 640)],
                                 accB.at[pl.ds(9360, 640)]))

    def norm_chunk(i, carry):
        da = pltpu.async_copy(dis.at[rowv.at[i]], d0, q0)
        db = pltpu.async_copy(dis.at[colv.at[i]], d1, q1)
        da.wait()
        db.wait()
        for g in range(_G // 16):
            sl = pl.ds(g * 16, 16)
            normv[i, sl] = d0[sl] * wv[i, sl] * d1[sl]
        return carry
    lax.fori_loop(0, _NCH, norm_chunk, 0)
    pl.when(c == 0)(lambda: pltpu.sync_copy(normv, norm3o.at[s]))
    plsc.subcore_barrier()

    bufs = (curA, accB)
    for h in range(3):
        cur = bufs[h % 2]
        tgt = bufs[1 - h % 2]
        if h > 0:
            both(lambda: pltpu.sync_copy(zsrc16.at[pl.ds(s * 624, 624)],
                                         tgt.at[pl.ds(s * 624, 624)]),
                 lambda: pltpu.sync_copy(zsrc16.at[pl.ds(9360, 640)],
                                         tgt.at[pl.ds(9360, 640)]))
            plsc.subcore_barrier()
        _small_chunk_loop(cur, tgt, rowv, colv, normv, g0, g1, s0, s1, q0, q1)
        plsc.subcore_barrier()
        pl.when(jnp.logical_and(c == 0, jnp.logical_not(last)))(
            lambda: pltpu.sync_copy(tgt.at[pl.ds(s * 624, 624)],
                                    outs.at[h].at[pl.ds(s * 624, 624)]))
        pl.when(jnp.logical_and(c == 0, last))(
            lambda: pltpu.sync_copy(tgt.at[pl.ds(9360, 640)],
                                    outs.at[h].at[pl.ds(9360, 640)]))
        plsc.subcore_barrier()


_l1_call = pl.kernel(
    _l1_body,
    out_type=[jax.ShapeDtypeStruct((3, _N, 16), jnp.float32),
              jax.ShapeDtypeStruct((_NT, _NCH, _G), jnp.float32)],
    mesh=plsc.VectorSubcoreMesh(core_axis_name="c", subcore_axis_name="s"),
    compiler_params=pltpu.CompilerParams(use_tc_tiling_on_sc=False),
    scratch_types=[
        pltpu.VMEM((_NCH, _G), jnp.int32),
        pltpu.VMEM((_NCH, _G), jnp.int32),
        pltpu.VMEM((_NCH, _G), jnp.float32),
        pltpu.VMEM((_NCH, _G), jnp.float32),
        pltpu.VMEM((_G, 16), jnp.float32),
        pltpu.VMEM((_G, 16), jnp.float32),
        pltpu.VMEM((_G,), jnp.float32),
        pltpu.VMEM((_G,), jnp.float32),
        pltpu.VMEM_SHARED((_N, 16), jnp.float32),
        pltpu.VMEM_SHARED((_N, 16), jnp.float32),
        pltpu.SemaphoreType.DMA,
        pltpu.SemaphoreType.DMA,
        pltpu.SemaphoreType.DMA,
        pltpu.SemaphoreType.DMA,
    ],
)


def _l3_body(ys, row3, col3, norm3, out,
             rowv, colv, normv, g0, g1,
             curA, accB, q0, q1, s0, s1):
    c = lax.axis_index("c")
    s = lax.axis_index("s")
    last = s == _NT - 1

    pltpu.sync_copy(row3.at[s], rowv)
    pltpu.sync_copy(col3.at[s], colv)
    pltpu.sync_copy(norm3.at[s], normv)

    def both(fn_main, fn_last):
        pl.when(jnp.logical_not(last))(fn_main)
        pl.when(last)(fn_last)

    both(lambda: pltpu.sync_copy(ys.at[3].at[pl.ds(s * 624, 624)],
                                 curA.at[pl.ds(s * 624, 624)]),
         lambda: pltpu.sync_copy(ys.at[3].at[pl.ds(9360, 640)],
                                 curA.at[pl.ds(9360, 640)]))
    bufs = (curA, accB)
    for h in range(3):
        cur = bufs[h % 2]
        tgt = bufs[1 - h % 2]
        yk = 2 - h
        both(lambda: pltpu.sync_copy(ys.at[yk].at[pl.ds(s * 624, 624)],
                                     tgt.at[pl.ds(s * 624, 624)]),
             lambda: pltpu.sync_copy(ys.at[yk].at[pl.ds(9360, 640)],
                                     tgt.at[pl.ds(9360, 640)]))
        plsc.subcore_barrier()
        _small_chunk_loop(cur, tgt, rowv, colv, normv, g0, g1, s0, s1, q0, q1)
        plsc.subcore_barrier()
    fin = bufs[1 - 2 % 2]
    pl.when(jnp.logical_and(c == 0, jnp.logical_not(last)))(
        lambda: pltpu.sync_copy(fin.at[pl.ds(s * 624, 624)],
                                out.at[pl.ds(s * 624, 624)]))
    pl.when(jnp.logical_and(c == 0, last))(
        lambda: pltpu.sync_copy(fin.at[pl.ds(9360, 640)],
                                out.at[pl.ds(9360, 640)]))


_l3_call = pl.kernel(
    _l3_body,
    out_type=jax.ShapeDtypeStruct((_N, 16), jnp.float32),
    mesh=plsc.VectorSubcoreMesh(core_axis_name="c", subcore_axis_name="s"),
    compiler_params=pltpu.CompilerParams(use_tc_tiling_on_sc=False),
    scratch_types=[
        pltpu.VMEM((_NCH, _G), jnp.int32),
        pltpu.VMEM((_NCH, _G), jnp.int32),
        pltpu.VMEM((_NCH, _G), jnp.float32),
        pltpu.VMEM((_G, 16), jnp.float32),
        pltpu.VMEM((_G, 16), jnp.float32),
        pltpu.VMEM_SHARED((_N, 16), jnp.float32),
        pltpu.VMEM_SHARED((_N, 16), jnp.float32),
        pltpu.SemaphoreType.DMA,
        pltpu.SemaphoreType.DMA,
        pltpu.SemaphoreType.DMA,
        pltpu.SemaphoreType.DMA,
    ],
)


def kernel(x, edge_index, edge_weight, W1, b1, W2, b2, W3, b3):
    N = x.shape[0]
    Np = 10240
    K1 = W1.shape[0]
    H = W1.shape[2]
    F = x.shape[1]

    row, col = edge_index[0], edge_index[1]

    def _meta(v):
        return jnp.pad(v.reshape(_NT, _E // _NT), ((0, 0), (0, _EWP - _E // _NT))
                       ).reshape(_NT, _NCH, _G)
    row3 = _meta(row)
    col3 = _meta(col)
    w3m = _meta(edge_weight)

    x16 = jnp.pad(x, ((0, 0), (0, 16 - F)))
    zsrc16 = jnp.zeros((_N, 16), jnp.float32)
    zsrc1d = jnp.zeros((_N,), jnp.float32)

    deg = _deg_call(zsrc1d, col3, w3m)
    dis = _dis_call(jnp.pad(deg, (0, _EWP - _N)).reshape(_EWP // _G, _G)
                    ).reshape(_EWP)[:_N]
    hops1, norm3 = _l1_call(x16, zsrc16, dis, row3, col3, w3m)

    X1 = jnp.concatenate([x16[None], hops1], axis=0)
    X1 = jnp.transpose(X1, (1, 0, 2)).reshape(_N, 64)
    X1 = jnp.pad(X1, ((0, Np - N), (0, 64)))
    W1s = jnp.pad(jnp.pad(W1, ((0, 0), (0, 16 - F), (0, 0))).reshape(64, H),
                  ((0, 64), (0, 0)))
    h1 = _mm(X1, W1s, b1, act=True)

    zsrc = jnp.zeros((_N, _CH), jnp.float32)

    def hop512(src):
        o = _hop512_call(src, zsrc, row3, col3, norm3)
        flat = jnp.transpose(o, (1, 0, 2)).reshape(_N, 512)
        return o, flat

    h1n = h1[:N]
    hops2 = [h1n]
    src = jnp.transpose(h1n.reshape(_N, _NQ, _CH), (1, 0, 2))
    for _ in range(K1 - 1):
        src, flat = hop512(src)
        hops2.append(flat)
    X2 = jnp.pad(jnp.concatenate(hops2, axis=1), ((0, Np - N), (0, 0)))
    W2s = W2.reshape(K1 * H, H)
    h2 = _mm(X2, W2s, b2, act=True)

    W3s = jnp.pad(jnp.pad(W3, ((0, 0), (0, 0), (0, 16 - F))
                  ).transpose(1, 0, 2).reshape(H, 64), ((0, 0), (0, 64)))
    b3p = jnp.pad(b3, (0, 128 - F))
    y = _mm(h2, W3s, b3p, act=False)[:N, :64]
    Y = jnp.transpose(y.reshape(_N, K1, 16), (1, 0, 2))
    t = _l3_call(Y, row3, col3, norm3)
    return t[:, :F]


# speedup vs baseline: 3.6598x; 1.0479x over previous
import functools

import jax
import jax.numpy as jnp
from jax import lax
from jax.experimental import pallas as pl
from jax.experimental.pallas import tpu as pltpu
from jax.experimental.pallas import tpu_sc as plsc

_MB = 512
_N = 10000
_E = 160000
_NT = 16
_G = 128
_EWP = 10240
_NCH = _EWP // _G


def _mm_body(x_ref, w_ref, b_ref, o_ref, *, act):
    acc = jnp.dot(x_ref[...], w_ref[...], preferred_element_type=jnp.float32)
    acc = acc + b_ref[...]
    if act:
        acc = jnp.where(acc > 0, acc, jnp.exp(jnp.minimum(acc, 0.0)) - 1.0)
    o_ref[...] = acc


def _mm(x, w, b, act):
    M, K = x.shape
    Nout = w.shape[1]
    return pl.pallas_call(
        functools.partial(_mm_body, act=act),
        grid=(M // _MB,),
        in_specs=[
            pl.BlockSpec((_MB, K), lambda i: (i, 0)),
            pl.BlockSpec((K, Nout), lambda i: (0, 0)),
            pl.BlockSpec((1, Nout), lambda i: (0, 0)),
        ],
        out_specs=pl.BlockSpec((_MB, Nout), lambda i: (i, 0)),
        out_shape=jax.ShapeDtypeStruct((M, Nout), jnp.float32),
    )(x, w, b.reshape(1, Nout))


_CH = 64
_NQ = 512 // _CH
_KV = _CH // 16


def _hop512_body(src, zsrc, row3, col3, norm3, out,
                 rowv, colv, normv, gbuf0, gbuf1, gbuf2, gbuf3, acc,
                 gsem0, gsem1, gsem2, gsem3, ssem0, ssem1, ssem2, ssem3):
    c = lax.axis_index("c")
    s = lax.axis_index("s")
    last = s == _NT - 1

    pltpu.sync_copy(row3.at[s], rowv)
    pltpu.sync_copy(col3.at[s], colv)
    pltpu.sync_copy(norm3.at[s], normv)

    gbufs = (gbuf0, gbuf1, gbuf2, gbuf3)
    gsems = (gsem0, gsem1, gsem2, gsem3)
    ssems = (ssem0, ssem1, ssem2, ssem3)

    def chunk_loop(q):
        def compute(i, buf):
            def edge16(eb, c2_):
                nvec = normv[i, pl.ds(eb * 16, 16)]
                for j in range(16):
                    nb = lax.broadcast(nvec[j], (16,))
                    e = eb * 16 + j
                    for k in range(_KV):
                        sl = pl.ds(k * 16, 16)
                        buf[e, sl] = buf[e, sl] * nb
                return c2_
            lax.fori_loop(0, _G // 16, edge16, 0)

        def body4(i4, carry):
            base = 4 * i4
            gds = [pltpu.async_copy(src.at[q].at[rowv.at[base + u]],
                                    gbufs[u], gsems[u]) for u in range(4)]
            sds = []
            for u in range(4):
                gds[u].wait()
                compute(base + u, gbufs[u])
                sds.append(pltpu.async_copy(
                    gbufs[u], acc.at[colv.at[base + u]], ssems[u], add=True))
            for sd in sds:
                sd.wait()
            return carry
        lax.fori_loop(0, _NCH // 4, body4, 0)

    for p in range(_NQ // 2):
        pl.when(jnp.logical_not(last))(
            lambda: pltpu.sync_copy(zsrc.at[pl.ds(s * 624, 624)],
                                    acc.at[pl.ds(s * 624, 624)]))
        pl.when(last)(
            lambda: pltpu.sync_copy(zsrc.at[pl.ds(9360, 640)],
                                    acc.at[pl.ds(9360, 640)]))
        plsc.subcore_barrier()
        for cv in range(2):
            pl.when(c == cv)(functools.partial(chunk_loop, (_NQ // 2) * cv + p))
        plsc.subcore_barrier()
        for cv in range(2):
            q = (_NQ // 2) * cv + p
            pl.when(jnp.logical_and(c == cv, jnp.logical_not(last)))(
                functools.partial(
                    lambda qq: pltpu.sync_copy(
                        acc.at[pl.ds(s * 624, 624)],
                        out.at[qq].at[pl.ds(s * 624, 624)]), q))
            pl.when(jnp.logical_and(c == cv, last))(
                functools.partial(
                    lambda qq: pltpu.sync_copy(
                        acc.at[pl.ds(9360, 640)],
                        out.at[qq].at[pl.ds(9360, 640)]), q))
        plsc.subcore_barrier()


_hop512_call = pl.kernel(
    _hop512_body,
    out_type=jax.ShapeDtypeStruct((_NQ, _N, _CH), jnp.float32),
    mesh=plsc.VectorSubcoreMesh(core_axis_name="c", subcore_axis_name="s"),
    compiler_params=pltpu.CompilerParams(use_tc_tiling_on_sc=False),
    scratch_types=[
        pltpu.VMEM((_NCH, _G), jnp.int32),
        pltpu.VMEM((_NCH, _G), jnp.int32),
        pltpu.VMEM((_NCH, _G), jnp.float32),
        pltpu.VMEM((_G, _CH), jnp.float32),
        pltpu.VMEM((_G, _CH), jnp.float32),
        pltpu.VMEM((_G, _CH), jnp.float32),
        pltpu.VMEM((_G, _CH), jnp.float32),
        pltpu.VMEM_SHARED((_N, _CH), jnp.float32),
    ] + [pltpu.SemaphoreType.DMA] * 8,
)


def _small_chunk_loop(cur, tgt, rowv, colv, normv, g0, g1, s0, s1, q0, q1):
    def compute(i, buf):
        def edge16(eb, c2_):
            nvec = normv[i, pl.ds(eb * 16, 16)]
            for j in range(16):
                nb = lax.broadcast(nvec[j], (16,))
                e = eb * 16 + j
                buf[e, :] = buf[e, :] * nb
            return c2_
        lax.fori_loop(0, _G // 16, edge16, 0)

    def body2(i2, carry):
        ia = 2 * i2
        ib = 2 * i2 + 1
        da = pltpu.async_copy(cur.at[rowv.at[ia]], g0, q0)
        db = pltpu.async_copy(cur.at[rowv.at[ib]], g1, q1)
        da.wait()
        compute(ia, g0)
        sa = pltpu.async_copy(g0, tgt.at[colv.at[ia]], s0, add=True)
        db.wait()
        compute(ib, g1)
        sb = pltpu.async_copy(g1, tgt.at[colv.at[ib]], s1, add=True)
        sa.wait()
        sb.wait()
        return carry
    lax.fori_loop(0, _NCH // 2, body2, 0)


def _dis_body(deg_ref, o_ref):
    d = deg_ref[...]
    o_ref[...] = jnp.where(d > 0, lax.rsqrt(jnp.maximum(d, 1e-30)), 0.0)


_dis_call = pl.pallas_call(
    _dis_body,
    out_shape=jax.ShapeDtypeStruct((_EWP // _G, _G), jnp.float32),
)


def _deg_body(zsrc1d, col3, w3, deg_o, colv, wv, acc1, q0, q1):
    c = lax.axis_index("c")
    s = lax.axis_index("s")
    last = s == _NT - 1

    pltpu.sync_copy(col3.at[s], colv)
    pltpu.sync_copy(w3.at[s], wv)

    def both(fn_main, fn_last):
        pl.when(jnp.logical_not(last))(fn_main)
        pl.when(last)(fn_last)

    both(lambda: pltpu.sync_copy(zsrc1d.at[pl.ds(s * 624, 624)],
                                 acc1.at[pl.ds(s * 624, 624)]),
         lambda: pltpu.sync_copy(zsrc1d.at[pl.ds(9360, 640)],
                                 acc1.at[pl.ds(9360, 640)]))
    plsc.subcore_barrier()

    def body2(i2, carry):
        sa = pltpu.async_copy(wv.at[2 * i2], acc1.at[colv.at[2 * i2]],
                              q0, add=True)
        sb = pltpu.async_copy(wv.at[2 * i2 + 1], acc1.at[colv.at[2 * i2 + 1]],
                              q1, add=True)
        sa.wait()
        sb.wait()
        return carry
    lax.fori_loop(0, _NCH // 2, body2, 0)
    plsc.subcore_barrier()
    pl.when(jnp.logical_and(c == 0, jnp.logical_not(last)))(
        lambda: pltpu.sync_copy(acc1.at[pl.ds(s * 624, 624)],
                                deg_o.at[pl.ds(s * 624, 624)]))
    pl.when(jnp.logical_and(c == 0, last))(
        lambda: pltpu.sync_copy(acc1.at[pl.ds(9360, 640)],
                                deg_o.at[pl.ds(9360, 640)]))


_deg_call = pl.kernel(
    _deg_body,
    out_type=jax.ShapeDtypeStruct((_N,), jnp.float32),
    mesh=plsc.VectorSubcoreMesh(core_axis_name="c", subcore_axis_name="s"),
    compiler_params=pltpu.CompilerParams(use_tc_tiling_on_sc=False),
    scratch_types=[
        pltpu.VMEM((_NCH, _G), jnp.int32),
        pltpu.VMEM((_NCH, _G), jnp.float32),
        pltpu.VMEM_SHARED((_N,), jnp.float32),
        pltpu.SemaphoreType.DMA,
        pltpu.SemaphoreType.DMA,
    ],
)


def _l1_body(x16, zsrc16, dis, row3, col3, w3, outs, norm3o,
             rowv, colv, wv, normv, g0, g1, d0, d1,
             curA, accB, q0, q1, s0, s1):
    c = lax.axis_index("c")
    s = lax.axis_index("s")
    last = s == _NT - 1

    pltpu.sync_copy(row3.at[s], rowv)
    pltpu.sync_copy(col3.at[s], colv)
    pltpu.sync_copy(w3.at[s], wv)

    def both(fn_main, fn_last):
        pl.when(jnp.logical_not(last))(fn_main)
        pl.when(last)(fn_last)

    both(lambda: pltpu.sync_copy(x16.at[pl.ds(s * 624, 624)],
                                 curA.at[pl.ds(s * 624, 624)]),
         lambda: pltpu.sync_copy(x16.at[pl.ds(9360, 640)],
                                 curA.at[pl.ds(9360, 640)]))
    both(lambda: pltpu.sync_copy(zsrc16.at[pl.ds(s * 624, 624)],
                                 accB.at[pl.ds(s * 624, 624)]),
         lambda: pltpu.sync_copy(zsrc16.at[pl.ds(9360, 640)],
                                 accB.at[pl.ds(9360, 640)]))

    def norm_chunk(i, carry):
        da = pltpu.async_copy(dis.at[rowv.at[i]], d0, q0)
        db = pltpu.async_copy(dis.at[colv.at[i]], d1, q1)
        da.wait()
        db.wait()
        for g in range(_G // 16):
            sl = pl.ds(g * 16, 16)
            normv[i, sl] = d0[sl] * wv[i, sl] * d1[sl]
        return carry
    lax.fori_loop(0, _NCH, norm_chunk, 0)
    pl.when(c == 0)(lambda: pltpu.sync_copy(normv, norm3o.at[s]))
    plsc.subcore_barrier()

    bufs = (curA, accB)
    for h in range(3):
        cur = bufs[h % 2]
        tgt = bufs[1 - h % 2]
        if h > 0:
            both(lambda: pltpu.sync_copy(zsrc16.at[pl.ds(s * 624, 624)],
                                         tgt.at[pl.ds(s * 624, 624)]),
                 lambda: pltpu.sync_copy(zsrc16.at[pl.ds(9360, 640)],
                                         tgt.at[pl.ds(9360, 640)]))
            plsc.subcore_barrier()
        _small_chunk_loop(cur, tgt, rowv, colv, normv, g0, g1, s0, s1, q0, q1)
        plsc.subcore_barrier()
        pl.when(jnp.logical_and(c == 0, jnp.logical_not(last)))(
            lambda: pltpu.sync_copy(tgt.at[pl.ds(s * 624, 624)],
                                    outs.at[h].at[pl.ds(s * 624, 624)]))
        pl.when(jnp.logical_and(c == 0, last))(
            lambda: pltpu.sync_copy(tgt.at[pl.ds(9360, 640)],
                                    outs.at[h].at[pl.ds(9360, 640)]))
        plsc.subcore_barrier()


_l1_call = pl.kernel(
    _l1_body,
    out_type=[jax.ShapeDtypeStruct((3, _N, 16), jnp.float32),
              jax.ShapeDtypeStruct((_NT, _NCH, _G), jnp.float32)],
    mesh=plsc.VectorSubcoreMesh(core_axis_name="c", subcore_axis_name="s"),
    compiler_params=pltpu.CompilerParams(use_tc_tiling_on_sc=False),
    scratch_types=[
        pltpu.VMEM((_NCH, _G), jnp.int32),
        pltpu.VMEM((_NCH, _G), jnp.int32),
        pltpu.VMEM((_NCH, _G), jnp.float32),
        pltpu.VMEM((_NCH, _G), jnp.float32),
        pltpu.VMEM((_G, 16), jnp.float32),
        pltpu.VMEM((_G, 16), jnp.float32),
        pltpu.VMEM((_G,), jnp.float32),
        pltpu.VMEM((_G,), jnp.float32),
        pltpu.VMEM_SHARED((_N, 16), jnp.float32),
        pltpu.VMEM_SHARED((_N, 16), jnp.float32),
        pltpu.SemaphoreType.DMA,
        pltpu.SemaphoreType.DMA,
        pltpu.SemaphoreType.DMA,
        pltpu.SemaphoreType.DMA,
    ],
)


def _l3_body(ys, row3, col3, norm3, out,
             rowv, colv, normv, g0, g1,
             curA, accB, q0, q1, s0, s1):
    c = lax.axis_index("c")
    s = lax.axis_index("s")
    last = s == _NT - 1

    pltpu.sync_copy(row3.at[s], rowv)
    pltpu.sync_copy(col3.at[s], colv)
    pltpu.sync_copy(norm3.at[s], normv)

    def both(fn_main, fn_last):
        pl.when(jnp.logical_not(last))(fn_main)
        pl.when(last)(fn_last)

    both(lambda: pltpu.sync_copy(ys.at[3].at[pl.ds(s * 624, 624)],
                                 curA.at[pl.ds(s * 624, 624)]),
         lambda: pltpu.sync_copy(ys.at[3].at[pl.ds(9360, 640)],
                                 curA.at[pl.ds(9360, 640)]))
    bufs = (curA, accB)
    for h in range(3):
        cur = bufs[h % 2]
        tgt = bufs[1 - h % 2]
        yk = 2 - h
        both(lambda: pltpu.sync_copy(ys.at[yk].at[pl.ds(s * 624, 624)],
                                     tgt.at[pl.ds(s * 624, 624)]),
             lambda: pltpu.sync_copy(ys.at[yk].at[pl.ds(9360, 640)],
                                     tgt.at[pl.ds(9360, 640)]))
        plsc.subcore_barrier()
        _small_chunk_loop(cur, tgt, rowv, colv, normv, g0, g1, s0, s1, q0, q1)
        plsc.subcore_barrier()
    fin = bufs[1 - 2 % 2]
    pl.when(jnp.logical_and(c == 0, jnp.logical_not(last)))(
        lambda: pltpu.sync_copy(fin.at[pl.ds(s * 624, 624)],
                                out.at[pl.ds(s * 624, 624)]))
    pl.when(jnp.logical_and(c == 0, last))(
        lambda: pltpu.sync_copy(fin.at[pl.ds(9360, 640)],
                                out.at[pl.ds(9360, 640)]))


_l3_call = pl.kernel(
    _l3_body,
    out_type=jax.ShapeDtypeStruct((_N, 16), jnp.float32),
    mesh=plsc.VectorSubcoreMesh(core_axis_name="c", subcore_axis_name="s"),
    compiler_params=pltpu.CompilerParams(use_tc_tiling_on_sc=False),
    scratch_types=[
        pltpu.VMEM((_NCH, _G), jnp.int32),
        pltpu.VMEM((_NCH, _G), jnp.int32),
        pltpu.VMEM((_NCH, _G), jnp.float32),
        pltpu.VMEM((_G, 16), jnp.float32),
        pltpu.VMEM((_G, 16), jnp.float32),
        pltpu.VMEM_SHARED((_N, 16), jnp.float32),
        pltpu.VMEM_SHARED((_N, 16), jnp.float32),
        pltpu.SemaphoreType.DMA,
        pltpu.SemaphoreType.DMA,
        pltpu.SemaphoreType.DMA,
        pltpu.SemaphoreType.DMA,
    ],
)


def kernel(x, edge_index, edge_weight, W1, b1, W2, b2, W3, b3):
    N = x.shape[0]
    Np = 10240
    K1 = W1.shape[0]
    H = W1.shape[2]
    F = x.shape[1]

    row, col = edge_index[0], edge_index[1]

    def _meta(v):
        return jnp.pad(v.reshape(_NT, _E // _NT), ((0, 0), (0, _EWP - _E // _NT))
                       ).reshape(_NT, _NCH, _G)
    row3 = _meta(row)
    col3 = _meta(col)
    w3m = _meta(edge_weight)

    x16 = jnp.pad(x, ((0, 0), (0, 16 - F)))
    zsrc16 = jnp.zeros((_N, 16), jnp.float32)
    zsrc1d = jnp.zeros((_N,), jnp.float32)

    deg = _deg_call(zsrc1d, col3, w3m)
    dis = _dis_call(jnp.pad(deg, (0, _EWP - _N)).reshape(_EWP // _G, _G)
                    ).reshape(_EWP)[:_N]
    hops1, norm3 = _l1_call(x16, zsrc16, dis, row3, col3, w3m)

    X1 = jnp.concatenate([x16[None], hops1], axis=0)
    X1 = jnp.transpose(X1, (1, 0, 2)).reshape(_N, 64)
    X1 = jnp.pad(X1, ((0, Np - N), (0, 64)))
    W1s = jnp.pad(jnp.pad(W1, ((0, 0), (0, 16 - F), (0, 0))).reshape(64, H),
                  ((0, 64), (0, 0)))
    h1 = _mm(X1, W1s, b1, act=True)

    zsrc = jnp.zeros((_N, _CH), jnp.float32)

    def hop512(src):
        o = _hop512_call(src, zsrc, row3, col3, norm3)
        flat = jnp.transpose(o, (1, 0, 2)).reshape(_N, 512)
        return o, flat

    h1n = h1[:N]
    hops2 = [h1n]
    src = jnp.transpose(h1n.reshape(_N, _NQ, _CH), (1, 0, 2))
    for _ in range(K1 - 1):
        src, flat = hop512(src)
        hops2.append(flat)
    X2 = jnp.pad(jnp.concatenate(hops2, axis=1), ((0, Np - N), (0, 0)))
    W2s = W2.reshape(K1 * H, H)
    h2 = _mm(X2, W2s, b2, act=True)

    W3s = jnp.pad(jnp.pad(W3, ((0, 0), (0, 0), (0, 16 - F))
                  ).transpose(1, 0, 2).reshape(H, 64), ((0, 0), (0, 64)))
    b3p = jnp.pad(b3, (0, 128 - F))
    y = _mm(h2, W3s, b3p, act=False)[:N, :64]
    Y = jnp.transpose(y.reshape(_N, K1, 16), (1, 0, 2))
    t = _l3_call(Y, row3, col3, norm3)
    return t[:, :F]


# speedup vs baseline: 5.6670x; 1.5484x over previous
import functools

import jax
import jax.numpy as jnp
from jax import lax
from jax.experimental import pallas as pl
from jax.experimental.pallas import tpu as pltpu
from jax.experimental.pallas import tpu_sc as plsc

_MB = 512
_N = 10000
_E = 160000
_NT = 16
_G = 128
_EWP = 10240
_NCH = _EWP // _G


def _mm_body(x_ref, w_ref, b_ref, o_ref, *, act):
    acc = jnp.dot(x_ref[...], w_ref[...], preferred_element_type=jnp.float32)
    acc = acc + b_ref[...]
    if act:
        acc = jnp.where(acc > 0, acc, jnp.exp(jnp.minimum(acc, 0.0)) - 1.0)
    o_ref[...] = acc


def _mm(x, w, b, act):
    M, K = x.shape
    Nout = w.shape[1]
    return pl.pallas_call(
        functools.partial(_mm_body, act=act),
        grid=(M // _MB,),
        in_specs=[
            pl.BlockSpec((_MB, K), lambda i: (i, 0)),
            pl.BlockSpec((K, Nout), lambda i: (0, 0)),
            pl.BlockSpec((1, Nout), lambda i: (0, 0)),
        ],
        out_specs=pl.BlockSpec((_MB, Nout), lambda i: (i, 0)),
        out_shape=jax.ShapeDtypeStruct((M, Nout), jnp.float32),
    )(x, w, b.reshape(1, Nout))


_CH = 64
_NQ = 512 // _CH
_KV = _CH // 16


def _hop512_body(src, zsrc, row3, col3, norm3, out,
                 rowv, colv, normv, gbuf0, gbuf1, gbuf2, gbuf3, acc,
                 gsem0, gsem1, gsem2, gsem3, ssem0, ssem1, ssem2, ssem3):
    c = lax.axis_index("c")
    s = lax.axis_index("s")
    last = s == _NT - 1

    pltpu.sync_copy(row3.at[s], rowv)
    pltpu.sync_copy(col3.at[s], colv)
    pltpu.sync_copy(norm3.at[s], normv)

    gbufs = (gbuf0, gbuf1, gbuf2, gbuf3)
    gsems = (gsem0, gsem1, gsem2, gsem3)
    ssems = (ssem0, ssem1, ssem2, ssem3)

    def chunk_loop(q):
        def compute(i, buf):
            def edge32(eb, c2_):
                nvec = normv[i, pl.ds(eb * 32, 16)]
                nvec2 = normv[i, pl.ds(eb * 32 + 16, 16)]
                for j in range(16):
                    nb = lax.broadcast(nvec[j], (16,))
                    nb2 = lax.broadcast(nvec2[j], (16,))
                    e = eb * 32 + j
                    for k in range(_KV):
                        sl = pl.ds(k * 16, 16)
                        buf[e, sl] = buf[e, sl] * nb
                    for k in range(_KV):
                        sl = pl.ds(k * 16, 16)
                        buf[e + 16, sl] = buf[e + 16, sl] * nb2
                return c2_
            lax.fori_loop(0, _G // 32, edge32, 0)

        def body4(i4, carry):
            base = 4 * i4
            gds = [pltpu.async_copy(src.at[q].at[rowv.at[base + u]],
                                    gbufs[u], gsems[u]) for u in range(4)]
            sds = []
            for u in range(4):
                gds[u].wait()
                compute(base + u, gbufs[u])
                sds.append(pltpu.async_copy(
                    gbufs[u], acc.at[colv.at[base + u]], ssems[u], add=True))
            for sd in sds:
                sd.wait()
            return carry
        lax.fori_loop(0, _NCH // 4, body4, 0)

    for p in range(_NQ // 2):
        pl.when(jnp.logical_not(last))(
            lambda: pltpu.sync_copy(zsrc.at[pl.ds(s * 624, 624)],
                                    acc.at[pl.ds(s * 624, 624)]))
        pl.when(last)(
            lambda: pltpu.sync_copy(zsrc.at[pl.ds(9360, 640)],
                                    acc.at[pl.ds(9360, 640)]))
        plsc.subcore_barrier()
        for cv in range(2):
            pl.when(c == cv)(functools.partial(chunk_loop, (_NQ // 2) * cv + p))
        plsc.subcore_barrier()
        for cv in range(2):
            q = (_NQ // 2) * cv + p
            pl.when(jnp.logical_and(c == cv, jnp.logical_not(last)))(
                functools.partial(
                    lambda qq: pltpu.sync_copy(
                        acc.at[pl.ds(s * 624, 624)],
                        out.at[qq].at[pl.ds(s * 624, 624)]), q))
            pl.when(jnp.logical_and(c == cv, last))(
                functools.partial(
                    lambda qq: pltpu.sync_copy(
                        acc.at[pl.ds(9360, 640)],
                        out.at[qq].at[pl.ds(9360, 640)]), q))
        plsc.subcore_barrier()


_hop512_call = pl.kernel(
    _hop512_body,
    out_type=jax.ShapeDtypeStruct((_NQ, _N, _CH), jnp.float32),
    mesh=plsc.VectorSubcoreMesh(core_axis_name="c", subcore_axis_name="s"),
    compiler_params=pltpu.CompilerParams(use_tc_tiling_on_sc=False),
    scratch_types=[
        pltpu.VMEM((_NCH, _G), jnp.int32),
        pltpu.VMEM((_NCH, _G), jnp.int32),
        pltpu.VMEM((_NCH, _G), jnp.float32),
        pltpu.VMEM((_G, _CH), jnp.float32),
        pltpu.VMEM((_G, _CH), jnp.float32),
        pltpu.VMEM((_G, _CH), jnp.float32),
        pltpu.VMEM((_G, _CH), jnp.float32),
        pltpu.VMEM_SHARED((_N, _CH), jnp.float32),
    ] + [pltpu.SemaphoreType.DMA] * 8,
)


def _small_chunk_loop(cur, tgt, rowv, colv, normv, g0, g1, s0, s1, q0, q1):
    def compute(i, buf):
        def edge16(eb, c2_):
            nvec = normv[i, pl.ds(eb * 16, 16)]
            for j in range(16):
                nb = lax.broadcast(nvec[j], (16,))
                e = eb * 16 + j
                buf[e, :] = buf[e, :] * nb
            return c2_
        lax.fori_loop(0, _G // 16, edge16, 0)

    def body2(i2, carry):
        ia = 2 * i2
        ib = 2 * i2 + 1
        da = pltpu.async_copy(cur.at[rowv.at[ia]], g0, q0)
        db = pltpu.async_copy(cur.at[rowv.at[ib]], g1, q1)
        da.wait()
        compute(ia, g0)
        sa = pltpu.async_copy(g0, tgt.at[colv.at[ia]], s0, add=True)
        db.wait()
        compute(ib, g1)
        sb = pltpu.async_copy(g1, tgt.at[colv.at[ib]], s1, add=True)
        sa.wait()
        sb.wait()
        return carry
    lax.fori_loop(0, _NCH // 2, body2, 0)


def _dis_body(deg_ref, o_ref):
    d = deg_ref[...]
    o_ref[...] = jnp.where(d > 0, lax.rsqrt(jnp.maximum(d, 1e-30)), 0.0)


_dis_call = pl.pallas_call(
    _dis_body,
    out_shape=jax.ShapeDtypeStruct((_EWP // _G, _G), jnp.float32),
)


def _deg_body(zsrc1d, col3, w3, deg_o, colv, wv, acc1, q0, q1):
    c = lax.axis_index("c")
    s = lax.axis_index("s")
    last = s == _NT - 1

    pltpu.sync_copy(col3.at[s], colv)
    pltpu.sync_copy(w3.at[s], wv)

    def both(fn_main, fn_last):
        pl.when(jnp.logical_not(last))(fn_main)
        pl.when(last)(fn_last)

    both(lambda: pltpu.sync_copy(zsrc1d.at[pl.ds(s * 624, 624)],
                                 acc1.at[pl.ds(s * 624, 624)]),
         lambda: pltpu.sync_copy(zsrc1d.at[pl.ds(9360, 640)],
                                 acc1.at[pl.ds(9360, 640)]))
    plsc.subcore_barrier()

    def body2(i2, carry):
        sa = pltpu.async_copy(wv.at[2 * i2], acc1.at[colv.at[2 * i2]],
                              q0, add=True)
        sb = pltpu.async_copy(wv.at[2 * i2 + 1], acc1.at[colv.at[2 * i2 + 1]],
                              q1, add=True)
        sa.wait()
        sb.wait()
        return carry
    lax.fori_loop(0, _NCH // 2, body2, 0)
    plsc.subcore_barrier()
    pl.when(jnp.logical_and(c == 0, jnp.logical_not(last)))(
        lambda: pltpu.sync_copy(acc1.at[pl.ds(s * 624, 624)],
                                deg_o.at[pl.ds(s * 624, 624)]))
    pl.when(jnp.logical_and(c == 0, last))(
        lambda: pltpu.sync_copy(acc1.at[pl.ds(9360, 640)],
                                deg_o.at[pl.ds(9360, 640)]))


_deg_call = pl.kernel(
    _deg_body,
    out_type=jax.ShapeDtypeStruct((_N,), jnp.float32),
    mesh=plsc.VectorSubcoreMesh(core_axis_name="c", subcore_axis_name="s"),
    compiler_params=pltpu.CompilerParams(use_tc_tiling_on_sc=False),
    scratch_types=[
        pltpu.VMEM((_NCH, _G), jnp.int32),
        pltpu.VMEM((_NCH, _G), jnp.float32),
        pltpu.VMEM_SHARED((_N,), jnp.float32),
        pltpu.SemaphoreType.DMA,
        pltpu.SemaphoreType.DMA,
    ],
)


def _l1_body(x16, zsrc16, dis, row3, col3, w3, outs, norm3o,
             rowv, colv, wv, normv, g0, g1, d0, d1,
             curA, accB, q0, q1, s0, s1):
    c = lax.axis_index("c")
    s = lax.axis_index("s")
    last = s == _NT - 1

    pltpu.sync_copy(row3.at[s], rowv)
    pltpu.sync_copy(col3.at[s], colv)
    pltpu.sync_copy(w3.at[s], wv)

    def both(fn_main, fn_last):
        pl.when(jnp.logical_not(last))(fn_main)
        pl.when(last)(fn_last)

    both(lambda: pltpu.sync_copy(x16.at[pl.ds(s * 624, 624)],
                                 curA.at[pl.ds(s * 624, 624)]),
         lambda: pltpu.sync_copy(x16.at[pl.ds(9360, 640)],
                                 curA.at[pl.ds(9360, 640)]))
    both(lambda: pltpu.sync_copy(zsrc16.at[pl.ds(s * 624, 624)],
                                 accB.at[pl.ds(s * 624, 624)]),
         lambda: pltpu.sync_copy(zsrc16.at[pl.ds(9360, 640)],
                                 accB.at[pl.ds(9360, 640)]))

    def norm_chunk(i, carry):
        da = pltpu.async_copy(dis.at[rowv.at[i]], d0, q0)
        db = pltpu.async_copy(dis.at[colv.at[i]], d1, q1)
        da.wait()
        db.wait()
        for g in range(_G // 16):
            sl = pl.ds(g * 16, 16)
            normv[i, sl] = d0[sl] * wv[i, sl] * d1[sl]
        return carry
    lax.fori_loop(0, _NCH, norm_chunk, 0)
    pl.when(c == 0)(lambda: pltpu.sync_copy(normv, norm3o.at[s]))
    plsc.subcore_barrier()

    bufs = (curA, accB)
    for h in range(3):
        cur = bufs[h % 2]
        tgt = bufs[1 - h % 2]
        if h > 0:
            both(lambda: pltpu.sync_copy(zsrc16.at[pl.ds(s * 624, 624)],
                                         tgt.at[pl.ds(s * 624, 624)]),
                 lambda: pltpu.sync_copy(zsrc16.at[pl.ds(9360, 640)],
                                         tgt.at[pl.ds(9360, 640)]))
            plsc.subcore_barrier()
        _small_chunk_loop(cur, tgt, rowv, colv, normv, g0, g1, s0, s1, q0, q1)
        plsc.subcore_barrier()
        pl.when(jnp.logical_and(c == 0, jnp.logical_not(last)))(
            lambda: pltpu.sync_copy(tgt.at[pl.ds(s * 624, 624)],
                                    outs.at[h].at[pl.ds(s * 624, 624)]))
        pl.when(jnp.logical_and(c == 0, last))(
            lambda: pltpu.sync_copy(tgt.at[pl.ds(9360, 640)],
                                    outs.at[h].at[pl.ds(9360, 640)]))
        plsc.subcore_barrier()


_l1_call = pl.kernel(
    _l1_body,
    out_type=[jax.ShapeDtypeStruct((3, _N, 16), jnp.float32),
              jax.ShapeDtypeStruct((_NT, _NCH, _G), jnp.float32)],
    mesh=plsc.VectorSubcoreMesh(core_axis_name="c", subcore_axis_name="s"),
    compiler_params=pltpu.CompilerParams(use_tc_tiling_on_sc=False),
    scratch_types=[
        pltpu.VMEM((_NCH, _G), jnp.int32),
        pltpu.VMEM((_NCH, _G), jnp.int32),
        pltpu.VMEM((_NCH, _G), jnp.float32),
        pltpu.VMEM((_NCH, _G), jnp.float32),
        pltpu.VMEM((_G, 16), jnp.float32),
        pltpu.VMEM((_G, 16), jnp.float32),
        pltpu.VMEM((_G,), jnp.float32),
        pltpu.VMEM((_G,), jnp.float32),
        pltpu.VMEM_SHARED((_N, 16), jnp.float32),
        pltpu.VMEM_SHARED((_N, 16), jnp.float32),
        pltpu.SemaphoreType.DMA,
        pltpu.SemaphoreType.DMA,
        pltpu.SemaphoreType.DMA,
        pltpu.SemaphoreType.DMA,
    ],
)


def _l3_body(ys, row3, col3, norm3, out,
             rowv, colv, normv, g0, g1,
             curA, accB, q0, q1, s0, s1):
    c = lax.axis_index("c")
    s = lax.axis_index("s")
    last = s == _NT - 1

    pltpu.sync_copy(row3.at[s], rowv)
    pltpu.sync_copy(col3.at[s], colv)
    pltpu.sync_copy(norm3.at[s], normv)

    def both(fn_main, fn_last):
        pl.when(jnp.logical_not(last))(fn_main)
        pl.when(last)(fn_last)

    both(lambda: pltpu.sync_copy(ys.at[3].at[pl.ds(s * 624, 624)],
                                 curA.at[pl.ds(s * 624, 624)]),
         lambda: pltpu.sync_copy(ys.at[3].at[pl.ds(9360, 640)],
                                 curA.at[pl.ds(9360, 640)]))
    bufs = (curA, accB)
    for h in range(3):
        cur = bufs[h % 2]
        tgt = bufs[1 - h % 2]
        yk = 2 - h
        both(lambda: pltpu.sync_copy(ys.at[yk].at[pl.ds(s * 624, 624)],
                                     tgt.at[pl.ds(s * 624, 624)]),
             lambda: pltpu.sync_copy(ys.at[yk].at[pl.ds(9360, 640)],
                                     tgt.at[pl.ds(9360, 640)]))
        plsc.subcore_barrier()
        _small_chunk_loop(cur, tgt, rowv, colv, normv, g0, g1, s0, s1, q0, q1)
        plsc.subcore_barrier()
    fin = bufs[1 - 2 % 2]
    pl.when(jnp.logical_and(c == 0, jnp.logical_not(last)))(
        lambda: pltpu.sync_copy(fin.at[pl.ds(s * 624, 624)],
                                out.at[pl.ds(s * 624, 624)]))
    pl.when(jnp.logical_and(c == 0, last))(
        lambda: pltpu.sync_copy(fin.at[pl.ds(9360, 640)],
                                out.at[pl.ds(9360, 640)]))


_l3_call = pl.kernel(
    _l3_body,
    out_type=jax.ShapeDtypeStruct((_N, 16), jnp.float32),
    mesh=plsc.VectorSubcoreMesh(core_axis_name="c", subcore_axis_name="s"),
    compiler_params=pltpu.CompilerParams(use_tc_tiling_on_sc=False),
    scratch_types=[
        pltpu.VMEM((_NCH, _G), jnp.int32),
        pltpu.VMEM((_NCH, _G), jnp.int32),
        pltpu.VMEM((_NCH, _G), jnp.float32),
        pltpu.VMEM((_G, 16), jnp.float32),
        pltpu.VMEM((_G, 16), jnp.float32),
        pltpu.VMEM_SHARED((_N, 16), jnp.float32),
        pltpu.VMEM_SHARED((_N, 16), jnp.float32),
        pltpu.SemaphoreType.DMA,
        pltpu.SemaphoreType.DMA,
        pltpu.SemaphoreType.DMA,
        pltpu.SemaphoreType.DMA,
    ],
)


def kernel(x, edge_index, edge_weight, W1, b1, W2, b2, W3, b3):
    N = x.shape[0]
    Np = 10240
    K1 = W1.shape[0]
    H = W1.shape[2]
    F = x.shape[1]

    row, col = edge_index[0], edge_index[1]

    def _meta(v):
        return jnp.pad(v.reshape(_NT, _E // _NT), ((0, 0), (0, _EWP - _E // _NT))
                       ).reshape(_NT, _NCH, _G)
    row3 = _meta(row)
    col3 = _meta(col)
    w3m = _meta(edge_weight)

    x16 = jnp.pad(x, ((0, 0), (0, 16 - F)))
    zsrc16 = jnp.zeros((_N, 16), jnp.float32)
    zsrc1d = jnp.zeros((_N,), jnp.float32)

    deg = _deg_call(zsrc1d, col3, w3m)
    dis = _dis_call(jnp.pad(deg, (0, _EWP - _N)).reshape(_EWP // _G, _G)
                    ).reshape(_EWP)[:_N]
    hops1, norm3 = _l1_call(x16, zsrc16, dis, row3, col3, w3m)

    X1 = jnp.concatenate([x16[None], hops1], axis=0)
    X1 = jnp.transpose(X1, (1, 0, 2)).reshape(_N, 64)
    X1 = jnp.pad(X1, ((0, Np - N), (0, 64)))
    W1s = jnp.pad(jnp.pad(W1, ((0, 0), (0, 16 - F), (0, 0))).reshape(64, H),
                  ((0, 64), (0, 0)))
    h1 = _mm(X1, W1s, b1, act=True)

    zsrc = jnp.zeros((_N, _CH), jnp.float32)

    def hop512(src):
        o = _hop512_call(src, zsrc, row3, col3, norm3)
        flat = jnp.transpose(o, (1, 0, 2)).reshape(_N, 512)
        return o, flat

    h1n = h1[:N]
    hops2 = [h1n]
    src = jnp.transpose(h1n.reshape(_N, _NQ, _CH), (1, 0, 2))
    for _ in range(K1 - 1):
        src, flat = hop512(src)
        hops2.append(flat)
    X2 = jnp.pad(jnp.concatenate(hops2, axis=1), ((0, Np - N), (0, 0)))
    W2s = W2.reshape(K1 * H, H)
    h2 = _mm(X2, W2s, b2, act=True)

    W3s = jnp.pad(jnp.pad(W3, ((0, 0), (0, 0), (0, 16 - F))
                  ).transpose(1, 0, 2).reshape(H, 64), ((0, 0), (0, 64)))
    b3p = jnp.pad(b3, (0, 128 - F))
    y = _mm(h2, W3s, b3p, act=False)[:N, :64]
    Y = jnp.transpose(y.reshape(_N, K1, 16), (1, 0, 2))
    t = _l3_call(Y, row3, col3, norm3)
    return t[:, :F]
